# baseline (speedup 1.0000x reference)
import jax
import jax.numpy as jnp
from jax import lax
from jax.experimental import pallas as pl
from jax.experimental.pallas import tpu as pltpu
from jax.experimental.pallas import tpu_sc as plsc

NUM_NODES = 50000
EMB_DIM = 64
NUM_EDGES = 800000
BATCH = 16384

NC = 2
NS = 16
L = 16

HALF = 25088
N_PAD = 2 * HALF
GARB = 512
SP_ROWS = HALF + GARB
ZROWS = SP_ROWS // NS
NRM = HALF // NS
NRM_C = 112
E = 128
DST_PAD = N_PAD

EPW = NUM_EDGES // (NC * NS)
MAXCH = 196
CAP_A = MAXCH * E + 2 * E
DUMP = MAXCH * E + E
PAD_EDGE = 1024

_mesh = lambda: plsc.VectorSubcoreMesh(
    core_axis_name="c", subcore_axis_name="s", num_cores=NC, num_subcores=NS)
_SC_PARAMS = pltpu.CompilerParams(
    needs_layout_passes=False, use_tc_tiling_on_sc=False)


def _dst_local(dst16, base, off):
    m = (dst16 >= base) & (dst16 < base + HALF)
    garb = HALF + ((off + lax.iota(jnp.int32, L)) & (GARB - 1))
    return jnp.where(m, dst16 - base, garb)


def _part_body(src_hbm, dst_hbm, esrc_hbm, edst_hbm, cnt_hbm,
               sv0, sv1, dv0, dv1, bsl, bdl, bsh, bdh, cnt_v,
               se0, se1, sfl, sfh):
    core = lax.axis_index("c")
    sub = lax.axis_index("s")
    w = sub * NC + core
    base_e = w * EPW
    svs, dvs, sems = (sv0, sv1), (dv0, dv1), (se0, se1)
    bufs = ((bsl, bdl, sfl), (bsh, bdh, sfh))

    def e_start(j, s):
        off = base_e + j * E
        pltpu.async_copy(src_hbm.at[pl.ds(off, E)], svs[s], sems[s])
        pltpu.async_copy(dst_hbm.at[pl.ds(off, E)], dvs[s], sems[s])

    def e_wait(s):
        pltpu.make_async_copy(src_hbm.at[pl.ds(0, E)], svs[s], sems[s]).wait()
        pltpu.make_async_copy(dst_hbm.at[pl.ds(0, E)], dvs[s], sems[s]).wait()

    def f_wait(h):
        bs, bd, sf = bufs[h]
        pltpu.make_async_copy(bs.at[pl.ds(0, E)],
                              esrc_hbm.at[h, w, pl.ds(DUMP, E)], sf).wait()
        pltpu.make_async_copy(bd.at[pl.ds(0, E)],
                              edst_hbm.at[h, w, pl.ds(DUMP, E)], sf).wait()

    def f_start(h, srcoff, nf):
        bs, bd, sf = bufs[h]
        pltpu.async_copy(bs.at[pl.ds(srcoff, E)],
                         esrc_hbm.at[h, w, pl.ds(nf * E, E)], sf)
        pltpu.async_copy(bd.at[pl.ds(srcoff, E)],
                         edst_hbm.at[h, w, pl.ds(nf * E, E)], sf)

    e_start(0, 0)
    e_start(1, 1)

    def _append(h, sv, dv, m, c, nf):
        bs, bd, _ = bufs[h]
        cs = plsc.cumsum(m.astype(jnp.int32))
        pos = c + cs - 1
        plsc.store_scatter(bs, [pos], sv, mask=m)
        plsc.store_scatter(bd, [pos], dv, mask=m)
        n = cs[L - 1]
        c_new = c + n
        cross1 = (c < E) & (c_new >= E)
        cross2 = c_new >= 2 * E

        @pl.when(cross1)
        def _():
            f_start(h, 0, nf)
            f_wait(h)

        @pl.when(cross2)
        def _():
            f_start(h, E, nf)
            f_wait(h)
            bs[pl.ds(0, L)] = bs[pl.ds(2 * E, L)]
            bd[pl.ds(0, L)] = bd[pl.ds(2 * E, L)]

        nf_new = nf + cross1.astype(jnp.int32) + cross2.astype(jnp.int32)
        c_fin = jnp.where(cross2, c_new - 2 * E, c_new)
        return c_fin, nf_new

    def _chunk(j, s, carry):
        cl, nl, ch, nh = carry
        e_wait(s)
        for k in range(E // L):
            sv = svs[s][pl.ds(k * L, L)]
            dv = dvs[s][pl.ds(k * L, L)]
            valid = (j * E + k * L + lax.iota(jnp.int32, L)) < EPW
            in_lo = dv < HALF
            cl, nl = _append(0, sv, dv, in_lo & valid, cl, nl)
            ch, nh = _append(1, sv, dv, (~in_lo) & valid, ch, nh)
        e_start(j + 2, s)
        return cl, nl, ch, nh

    def _pair(p, carry):
        carry = _chunk(2 * p, 0, carry)
        carry = _chunk(2 * p + 1, 1, carry)
        return carry
    carry = lax.fori_loop(0, MAXCH // 2, _pair, (0, 0, 0, 0))
    cl, nl, ch, nh = carry
    e_wait(0)
    e_wait(1)

    zero16 = jnp.zeros((L,), jnp.int32)
    pad16 = jnp.full((L,), DST_PAD, jnp.int32)

    def _finish(h, c, nf):
        bs, bd, _ = bufs[h]

        def _pad(i, cc):
            bs[pl.ds(c + i * L, L)] = zero16
            bd[pl.ds(c + i * L, L)] = pad16
            return cc
        lax.fori_loop(0, 9, _pad, 0)
        has_part = (c & (E - 1)) != 0
        blk = (c // E) * E

        @pl.when(has_part)
        def _():
            f_start(h, blk, nf)
            f_wait(h)
        nf2 = nf + has_part.astype(jnp.int32)
        odd = (nf2 & 1) != 0
        al8 = ((c + 7) // 8) * 8

        @pl.when(odd)
        def _():
            f_start(h, al8, nf2)
            f_wait(h)
        nf3 = nf2 + odd.astype(jnp.int32)
        return nf3

    nf_lo = _finish(0, cl, nl)
    nf_hi = _finish(1, ch, nh)
    cnt_v[pl.ds(0, L)] = jnp.where(
        lax.iota(jnp.int32, L) == 0, nf_lo, nf_hi)
    pltpu.sync_copy(cnt_v, cnt_hbm.at[w])


def _make_conv_body(with_deg):
    def body(emb_hbm, esrc_hbm, edst_hbm, cnt_hbm, *rest):
        if with_deg:
            (out_hbm, inv_hbm, agg_sp, deg_sp,
             src0, src1, dst0, dst1, idx0, idx1, row0, row1,
             one_v, zer1, nrm_v, wrk_v, cnt_v,
             se0, se1, sg0, sg1, ss0, ss1, sd0, sd1) = rest
        else:
            (inv_hbm, out_hbm, agg_sp,
             src0, src1, dst0, dst1, idx0, idx1, row0, row1,
             zer1, nrm_v, wrk_v, cnt_v,
             se0, se1, sg0, sg1, ss0, ss1) = rest
            deg_sp = one_v = sd0 = sd1 = None
        srcs, dsts, idxs, rows = (src0, src1), (dst0, dst1), (idx0, idx1), (row0, row1)
        sem_e, sem_g, sem_s, sem_d = (se0, se1), (sg0, sg1), (ss0, ss1), (sd0, sd1)

        core = lax.axis_index("c")
        sub = lax.axis_index("s")
        base = core * HALF

        def _zrow(r, c):
            for k in range(EMB_DIM // L):
                row0[r, pl.ds(k * L, L)] = jnp.zeros((L,), jnp.float32)
            return c
        lax.fori_loop(0, E, _zrow, 0)
        z0 = sub * ZROWS
        for q in range(ZROWS // E):
            pltpu.sync_copy(row0, agg_sp.at[pl.ds(z0 + q * E, E)])
        pltpu.sync_copy(row0.at[pl.ds(0, ZROWS - (ZROWS // E) * E)],
                        agg_sp.at[pl.ds(z0 + (ZROWS // E) * E,
                                        ZROWS - (ZROWS // E) * E)])
        if with_deg:
            def _z1(i, c):
                zer1[pl.ds(i * L, L)] = jnp.zeros((L,), jnp.float32)
                return c
            lax.fori_loop(0, 160 // L, _z1, 0)
            for q in range(ZROWS // 160):
                pltpu.sync_copy(zer1, deg_sp.at[pl.ds(z0 + q * 160, 160)])

            def _ones(i, c):
                one_v[pl.ds(i * L, L)] = jnp.full((L,), 1.0, jnp.float32)
                return c
            lax.fori_loop(0, E // L, _ones, 0)
        plsc.subcore_barrier()

        def _region(rg):
            def _e_start(j, s):
                pltpu.async_copy(
                    esrc_hbm.at[core, rg, pl.ds(j * E, E)], srcs[s], sem_e[s])
                pltpu.async_copy(
                    edst_hbm.at[core, rg, pl.ds(j * E, E)], dsts[s], sem_e[s])

            def _e_wait(s):
                pltpu.make_async_copy(
                    esrc_hbm.at[0, 0, pl.ds(0, E)], srcs[s], sem_e[s]).wait()
                pltpu.make_async_copy(
                    edst_hbm.at[0, 0, pl.ds(0, E)], dsts[s], sem_e[s]).wait()

            def _x(j, s):
                for k in range(E // L):
                    d = dsts[s][pl.ds(k * L, L)]
                    idxs[s][pl.ds(k * L, L)] = _dst_local(
                        d, base, j * E + k * L)

            def _g_start(s):
                pltpu.async_copy(emb_hbm.at[srcs[s]], rows[s], sem_g[s])

            def _g_wait(s):
                pltpu.make_async_copy(
                    emb_hbm.at[srcs[s]], rows[s], sem_g[s]).wait()

            def _s_start(s):
                pltpu.async_copy(rows[s], agg_sp.at[idxs[s]], sem_s[s],
                                 add=True)
                if with_deg:
                    pltpu.async_copy(one_v, deg_sp.at[idxs[s]], sem_d[s],
                                     add=True)

            def _s_wait(s):
                pltpu.make_async_copy(
                    rows[s], agg_sp.at[idxs[s]], sem_s[s]).wait()
                if with_deg:
                    pltpu.make_async_copy(
                        one_v, deg_sp.at[idxs[s]], sem_d[s]).wait()

            pltpu.sync_copy(cnt_hbm.at[rg], cnt_v)
            v = cnt_v[pl.ds(0, L)]
            nf = jnp.where(core == 0, v[0], v[1])

            @pl.when(nf > 0)
            def _():
                _e_start(0, 0)
                _e_wait(0)
                _x(0, 0)
                _g_start(0)
                _e_start(1, 1)
                _e_wait(1)
                _x(1, 1)
                _g_start(1)
                _g_wait(0)
                _e_start(2, 0)
                _s_start(0)

                def _full(j, s):
                    o = 1 - s
                    _e_wait(s)
                    _s_wait(s)
                    _x(j, s)
                    _g_start(s)
                    _g_wait(o)
                    _e_start(j + 1, o)
                    _s_start(o)

                def _pair(p, c):
                    _full(2 + 2 * p, 0)
                    _full(3 + 2 * p, 1)
                    return c
                lax.fori_loop(0, (nf - 2) // 2, _pair, 0)
                _e_wait(0)
                _g_wait(1)
                _s_wait(0)
                _s_start(1)
                _s_wait(1)

        _region(2 * sub)
        _region(2 * sub + 1)
        plsc.subcore_barrier()

        r0 = sub * NRM
        for q in range(NRM // NRM_C):
            rq = r0 + q * NRM_C
            if with_deg:
                pltpu.sync_copy(deg_sp.at[pl.ds(rq, NRM_C)], wrk_v)

                def _inv(i, c):
                    v = wrk_v[pl.ds(i * L, L)]
                    wrk_v[pl.ds(i * L, L)] = 1.0 / jnp.maximum(v, 1.0)
                    return c
                lax.fori_loop(0, NRM_C // L, _inv, 0)
                pltpu.sync_copy(wrk_v, inv_hbm.at[pl.ds(base + rq, NRM_C)])
            else:
                pltpu.sync_copy(inv_hbm.at[pl.ds(base + rq, NRM_C)], wrk_v)
            pltpu.sync_copy(agg_sp.at[pl.ds(rq, NRM_C)], nrm_v)

            def _scale(r, c):
                s = plsc.load_gather(wrk_v, [jnp.full((L,), r, jnp.int32)])
                for k in range(EMB_DIM // L):
                    nrm_v[r, pl.ds(k * L, L)] = nrm_v[r, pl.ds(k * L, L)] * s
                return c
            lax.fori_loop(0, NRM_C, _scale, 0)
            pltpu.sync_copy(nrm_v, out_hbm.at[pl.ds(base + rq, NRM_C)])
    return body


def _gather_body(a_hbm, g1_hbm, g2_hbm, p_hbm, au_hbm, pa_hbm,
                 oa_hbm, op_hbm, idx_v, acc_v, tmp_v, sem):
    core = lax.axis_index("c")
    sub = lax.axis_index("s")
    wid = sub * NC + core
    per_w = BATCH // (NC * NS)

    def _acc_add(r, c):
        for k in range(EMB_DIM // L):
            acc_v[r, pl.ds(k * L, L)] = (
                acc_v[r, pl.ds(k * L, L)] + tmp_v[r, pl.ds(k * L, L)])
        return c

    for q in range(per_w // E):
        b0 = wid * per_w + q * E
        pltpu.sync_copy(au_hbm.at[pl.ds(b0, E)], idx_v)
        pltpu.async_copy(a_hbm.at[idx_v], acc_v, sem).wait()
        pltpu.async_copy(g1_hbm.at[idx_v], tmp_v, sem).wait()
        lax.fori_loop(0, E, _acc_add, 0)
        pltpu.async_copy(g2_hbm.at[idx_v], tmp_v, sem).wait()
        lax.fori_loop(0, E, _acc_add, 0)
        pltpu.sync_copy(acc_v, oa_hbm.at[pl.ds(b0, E)])

        pltpu.sync_copy(pa_hbm.at[pl.ds(b0, E)], idx_v)
        pltpu.async_copy(p_hbm.at[idx_v], tmp_v, sem).wait()
        pltpu.sync_copy(tmp_v, op_hbm.at[pl.ds(b0, E)])


def _predict_body(a_ref, p_ref, o_ref):
    o_ref[...] = jax.nn.sigmoid(jnp.sum(a_ref[...] * p_ref[...], axis=1))


@jax.jit
def _run(authors, papers, src, dst, author_emb, paper_emb):
    f32 = jnp.float32
    i32 = jnp.int32

    src = jnp.concatenate([src, jnp.zeros((PAD_EDGE,), i32)])
    dst = jnp.concatenate([dst, jnp.full((PAD_EDGE,), DST_PAD, i32)])

    part = pl.kernel(
        _part_body,
        out_type=[
            jax.ShapeDtypeStruct((NC, NC * NS, CAP_A), i32),
            jax.ShapeDtypeStruct((NC, NC * NS, CAP_A), i32),
            jax.ShapeDtypeStruct((NC * NS, L), i32),
        ],
        mesh=_mesh(),
        compiler_params=_SC_PARAMS,
        scratch_types=(
            [pltpu.VMEM((E,), i32)] * 4
            + [pltpu.VMEM((400,), i32)] * 4
            + [pltpu.VMEM((L,), i32)]
            + [pltpu.SemaphoreType.DMA] * 4
        ),
    )
    esrc, edst, cnts = part(src, dst)

    def pipe_scratch():
        return [
            pltpu.VMEM((E,), i32), pltpu.VMEM((E,), i32),
            pltpu.VMEM((E,), i32), pltpu.VMEM((E,), i32),
            pltpu.VMEM((E,), i32), pltpu.VMEM((E,), i32),
            pltpu.VMEM((E, EMB_DIM), f32), pltpu.VMEM((E, EMB_DIM), f32),
        ]

    conv1 = pl.kernel(
        _make_conv_body(True),
        out_type=[
            jax.ShapeDtypeStruct((N_PAD, EMB_DIM), f32),
            jax.ShapeDtypeStruct((N_PAD,), f32),
        ],
        mesh=_mesh(),
        compiler_params=_SC_PARAMS,
        scratch_types=(
            [pltpu.VMEM_SHARED((SP_ROWS, EMB_DIM), f32),
             pltpu.VMEM_SHARED((SP_ROWS,), f32)]
            + pipe_scratch()
            + [pltpu.VMEM((E,), f32),
               pltpu.VMEM((160,), f32),
               pltpu.VMEM((NRM_C, EMB_DIM), f32),
               pltpu.VMEM((NRM_C,), f32),
               pltpu.VMEM((L,), i32)]
            + [pltpu.SemaphoreType.DMA] * 8
        ),
    )
    g1, inv = conv1(author_emb, esrc, edst, cnts)

    conv2 = pl.kernel(
        _make_conv_body(False),
        out_type=jax.ShapeDtypeStruct((N_PAD, EMB_DIM), f32),
        mesh=_mesh(),
        compiler_params=_SC_PARAMS,
        scratch_types=(
            [pltpu.VMEM_SHARED((SP_ROWS, EMB_DIM), f32)]
            + pipe_scratch()
            + [pltpu.VMEM((160,), f32),
               pltpu.VMEM((NRM_C, EMB_DIM), f32),
               pltpu.VMEM((NRM_C,), f32),
               pltpu.VMEM((L,), i32)]
            + [pltpu.SemaphoreType.DMA] * 6
        ),
    )
    g2 = conv2(g1, esrc, edst, cnts, inv)

    gather_kernel = pl.kernel(
        _gather_body,
        out_type=[
            jax.ShapeDtypeStruct((BATCH, EMB_DIM), f32),
            jax.ShapeDtypeStruct((BATCH, EMB_DIM), f32),
        ],
        mesh=_mesh(),
        compiler_params=_SC_PARAMS,
        scratch_types=[
            pltpu.VMEM((E,), i32),
            pltpu.VMEM((E, EMB_DIM), f32),
            pltpu.VMEM((E, EMB_DIM), f32),
            pltpu.SemaphoreType.DMA,
        ],
    )
    la, lp = gather_kernel(author_emb, g1, g2, paper_emb, authors, papers)

    blk = 2048
    pred = pl.pallas_call(
        _predict_body,
        grid=(BATCH // blk,),
        in_specs=[
            pl.BlockSpec((blk, EMB_DIM), lambda i: (i, 0)),
            pl.BlockSpec((blk, EMB_DIM), lambda i: (i, 0)),
        ],
        out_specs=pl.BlockSpec((blk,), lambda i: (i,)),
        out_shape=jax.ShapeDtypeStruct((BATCH,), f32),
    )(la, lp)
    return pred, la, lp


def kernel(authors, papers, edge_index, author_emb, paper_emb):
    authors = authors.astype(jnp.int32)
    papers = papers.astype(jnp.int32)
    src = edge_index[0].astype(jnp.int32)
    dst = edge_index[1].astype(jnp.int32)
    return _run(authors, papers, src, dst, author_emb, paper_emb)
